# Initial kernel scaffold; baseline (speedup 1.0000x reference)
import jax
import jax.numpy as jnp
from jax.experimental import pallas as pl


def kernel(notation_ids, emb_table, W, b):
    raise NotImplementedError("write your pallas kernel here")


# speedup vs baseline: 1.3070x; 1.3070x over previous
import functools

import jax
import jax.numpy as jnp
from jax import lax
from jax.experimental import pallas as pl
from jax.experimental.pallas import tpu as pltpu
from jax.experimental.pallas import tpu_sc as plsc

VOCAB = 1000
D = 512
B_TOTAL = 4096 * 20


def _project_body(emb_ref, w_ref, b_ref, out_ref):
    p = lax.dot_general(
        emb_ref[...], w_ref[...],
        dimension_numbers=(((1,), (1,)), ((), ())),
        preferred_element_type=jnp.float32,
    )
    out_ref[...] = p + b_ref[...]


def _project_table(emb_table, W, b):
    return pl.pallas_call(
        _project_body,
        out_shape=jax.ShapeDtypeStruct((VOCAB, D), jnp.float32),
    )(emb_table, W, b.reshape(1, D))


_NW = 32
_B_PER_W = B_TOTAL // _NW
_CHUNK = 64
_NCHUNK = _B_PER_W // _CHUNK


def _make_gather():
    mesh = plsc.VectorSubcoreMesh(core_axis_name="c", subcore_axis_name="s")

    @functools.partial(
        pl.kernel,
        mesh=mesh,
        out_type=jax.ShapeDtypeStruct((B_TOTAL, D), jnp.float32),
        scratch_types=[
            pltpu.VMEM((_B_PER_W,), jnp.int32),
            pltpu.VMEM((_CHUNK, D), jnp.float32),
            pltpu.SemaphoreType.DMA,
        ],
    )
    def gather_kernel(table_hbm, idx_hbm, out_hbm, idx_v, rows_v, gsem):
        wid = lax.axis_index("s") * 2 + lax.axis_index("c")
        base = wid * _B_PER_W
        pltpu.sync_copy(idx_hbm.at[pl.ds(base, _B_PER_W)], idx_v)

        def body(g, _):
            pltpu.async_copy(
                table_hbm.at[idx_v.at[pl.ds(g * _CHUNK, _CHUNK)]],
                rows_v, gsem).wait()
            pltpu.sync_copy(
                rows_v, out_hbm.at[pl.ds(base + g * _CHUNK, _CHUNK)])
            return 0

        lax.fori_loop(0, _NCHUNK, body, 0)

    return gather_kernel


def kernel(notation_ids, emb_table, W, b):
    P = _project_table(emb_table, W, b)
    ids_flat = notation_ids.reshape(-1).astype(jnp.int32)
    out_flat = _make_gather()(P, ids_flat)
    return out_flat.reshape(notation_ids.shape + (D,))


# speedup vs baseline: 1.3539x; 1.0358x over previous
import functools

import jax
import jax.numpy as jnp
from jax import lax
from jax.experimental import pallas as pl
from jax.experimental.pallas import tpu as pltpu
from jax.experimental.pallas import tpu_sc as plsc

VOCAB = 1000
D = 512
B_TOTAL = 4096 * 20


def _project_body(emb_ref, w_ref, b_ref, out_ref):
    p = lax.dot_general(
        emb_ref[...], w_ref[...],
        dimension_numbers=(((1,), (1,)), ((), ())),
        preferred_element_type=jnp.float32,
    )
    out_ref[...] = p + b_ref[...]


def _project_table(emb_table, W, b):
    return pl.pallas_call(
        _project_body,
        out_shape=jax.ShapeDtypeStruct((VOCAB, D), jnp.float32),
    )(emb_table, W, b.reshape(1, D))


_NW = 32
_B_PER_W = B_TOTAL // _NW
_CHUNK = 64
_NCHUNK = _B_PER_W // _CHUNK


def _make_gather():
    mesh = plsc.VectorSubcoreMesh(core_axis_name="c", subcore_axis_name="s")

    @functools.partial(
        pl.kernel,
        mesh=mesh,
        out_type=jax.ShapeDtypeStruct((B_TOTAL, D), jnp.float32),
        scratch_types=[
            pltpu.VMEM((_B_PER_W,), jnp.int32),
            pltpu.VMEM((2, _CHUNK, D), jnp.float32),
            pltpu.SemaphoreType.DMA,
            pltpu.SemaphoreType.DMA,
        ],
    )
    def gather_kernel(table_hbm, idx_hbm, out_hbm, idx_v, rows_v, gsem, wsem):
        wid = lax.axis_index("s") * 2 + lax.axis_index("c")
        base = wid * _B_PER_W
        pltpu.sync_copy(idx_hbm.at[pl.ds(base, _B_PER_W)], idx_v)

        def gcopy(g, slot):
            return pltpu.make_async_copy(
                table_hbm.at[idx_v.at[pl.ds(g * _CHUNK, _CHUNK)]],
                rows_v.at[slot], gsem)

        def wcopy(g, slot):
            return pltpu.make_async_copy(
                rows_v.at[slot],
                out_hbm.at[pl.ds(base + g * _CHUNK, _CHUNK)], wsem)

        gcopy(0, 0).start()
        gcopy(0, 0).wait()
        wcopy(0, 0).start()
        gcopy(1, 1).start()

        def body(g, _):
            slot = g % 2
            gcopy(g, slot).wait()
            wcopy(g, slot).start()
            wcopy(g - 1, 1 - slot).wait()
            gcopy(g + 1, 1 - slot).start()
            return 0

        lax.fori_loop(1, _NCHUNK - 1, body, 0)

        g_last = _NCHUNK - 1
        s_last = g_last % 2
        gcopy(g_last, s_last).wait()
        wcopy(g_last, s_last).start()
        wcopy(g_last - 1, 1 - s_last).wait()
        wcopy(g_last, s_last).wait()

    return gather_kernel


def kernel(notation_ids, emb_table, W, b):
    P = _project_table(emb_table, W, b)
    ids_flat = notation_ids.reshape(-1).astype(jnp.int32)
    out_flat = _make_gather()(P, ids_flat)
    return out_flat.reshape(notation_ids.shape + (D,))
